# baseline (speedup 1.0000x reference)
import functools

import jax
import jax.numpy as jnp
from jax import lax
from jax.experimental import pallas as pl
from jax.experimental.pallas import tpu as pltpu
from jax.experimental.pallas import tpu_sc as plsc

B, L, D = 16384, 20, 128
H, O = 1024, 1000

NCH = 8
BCH = B // NCH

NC, NS = 2, 16
NW = NC * NS
ROWS_PER_W = BCH // NW
CB = 16
NCHUNK = ROWS_PER_W // CB
CROWS = CB * L
NLANE = D // 16


def _pool_body(off, xf_hbm, e_hbm, out_hbm,
               idx0, idx1, rows0, rows1, acc_v,
               sem0, sem1, isem0, isem1):
    wid = lax.axis_index("s") * NC + lax.axis_index("c")
    base = wid * ROWS_PER_W
    idx = (idx0, idx1)
    rows = (rows0, rows1)
    sem = (sem0, sem1)
    isem = (isem0, isem1)

    def xsrc(c):
        return xf_hbm.at[pl.ds((off + base + c * CB) * L, CROWS)]

    def stage_idx(c):
        pltpu.async_copy(xsrc(c), idx[c % 2], isem[c % 2])

    def gather(c):
        b = c % 2
        pltpu.make_async_copy(xsrc(c), idx[b], isem[b]).wait()
        pltpu.async_copy(e_hbm.at[idx[b]], rows[b], sem[b])

    def accum(c):
        b = c % 2
        pltpu.make_async_copy(e_hbm.at[idx[b]], rows[b], sem[b]).wait()
        rv = rows[b]

        def row(r, carry):
            accs = [rv[r * L, pl.ds(d * 16, 16)] for d in range(NLANE)]
            for l in range(1, L):
                for d in range(NLANE):
                    accs[d] = accs[d] + rv[r * L + l, pl.ds(d * 16, 16)]
            for d in range(NLANE):
                acc_v[r, pl.ds(d * 16, 16)] = accs[d]
            return carry

        lax.fori_loop(0, CB, row, 0)
        pltpu.sync_copy(acc_v, out_hbm.at[pl.ds(base + c * CB, CB)])

    stage_idx(0)
    gather(0)
    stage_idx(1)
    for c in range(NCHUNK):
        if c + 1 < NCHUNK:
            gather(c + 1)
        accum(c)
        if c + 2 < NCHUNK:
            stage_idx(c + 2)


def _make_pool(ci):
    return functools.partial(
        pl.kernel,
        out_type=jax.ShapeDtypeStruct((BCH, D), jnp.float32),
        mesh=plsc.VectorSubcoreMesh(core_axis_name="c", subcore_axis_name="s"),
        scratch_types=[
            pltpu.VMEM((CROWS,), jnp.int32),
            pltpu.VMEM((CROWS,), jnp.int32),
            pltpu.VMEM((CROWS, D), jnp.float32),
            pltpu.VMEM((CROWS, D), jnp.float32),
            pltpu.VMEM((CB, D), jnp.float32),
            pltpu.SemaphoreType.DMA,
            pltpu.SemaphoreType.DMA,
            pltpu.SemaphoreType.DMA,
            pltpu.SemaphoreType.DMA,
        ],
    )(functools.partial(_pool_body, ci * BCH))


_POOLS = [_make_pool(ci) for ci in range(NCH)]

BM = 512
NBLK = BCH // BM


def _mlp_body(x_ref, hs_ref, w1t_ref, b1_ref, w2t_ref, b2_ref, o_ref):
    cnt = jnp.sum((x_ref[...] != 0).astype(jnp.float32), axis=1, keepdims=True)
    h = hs_ref[...] / jnp.maximum(cnt, 1.0)
    h1t = lax.dot_general(w1t_ref[...], h, (((1,), (1,)), ((), ())),
                          preferred_element_type=jnp.float32)
    h1t = jnp.maximum(h1t + b1_ref[...], 0.0)
    o_ref[...] = jnp.dot(w2t_ref[...], h1t,
                         preferred_element_type=jnp.float32) + b2_ref[...]


def _mlp_chunk(ci, x, hs, W1t, b1c, W2t, b2c, acc):
    base = ci * NBLK
    in_specs = [
        pl.BlockSpec((BM, L), lambda i: (base + i, 0)),
        pl.BlockSpec((BM, D), lambda i: (i, 0)),
        pl.BlockSpec((H, D), lambda i: (0, 0)),
        pl.BlockSpec((H, 1), lambda i: (0, 0)),
        pl.BlockSpec((O, H), lambda i: (0, 0)),
        pl.BlockSpec((O, 1), lambda i: (0, 0)),
    ]
    args = [x, hs, W1t, b1c, W2t, b2c]
    kwargs = {}
    body = _mlp_body
    if acc is not None:
        in_specs.append(pl.BlockSpec(memory_space=pl.ANY))
        args.append(acc)
        kwargs["input_output_aliases"] = {6: 0}
        body = lambda x_r, hs_r, w1_r, b1_r, w2_r, b2_r, a_r, o_r: (
            _mlp_body(x_r, hs_r, w1_r, b1_r, w2_r, b2_r, o_r))
    return pl.pallas_call(
        body,
        grid=(NBLK,),
        in_specs=in_specs,
        out_specs=pl.BlockSpec((O, BM), lambda i: (0, base + i)),
        out_shape=jax.ShapeDtypeStruct((O, B), jnp.float32),
        **kwargs,
    )(*args)


def kernel(x, E, W1, b1, W2, b2):
    x = x.astype(jnp.int32)
    xf = x.reshape(-1)
    W1t = W1.T
    W2t = W2.T
    b1c = b1.reshape(H, 1)
    b2c = b2.reshape(O, 1)
    hs = [_POOLS[ci](xf, E) for ci in range(NCH)]
    outt = None
    for ci in range(NCH):
        outt = _mlp_chunk(ci, x, hs[ci], W1t, b1c, W2t, b2c, outt)
    return outt.T


# speedup vs baseline: 1.2086x; 1.2086x over previous
import functools

import jax
import jax.numpy as jnp
from jax import lax
from jax.experimental import pallas as pl
from jax.experimental.pallas import tpu as pltpu
from jax.experimental.pallas import tpu_sc as plsc

B, L, D = 16384, 20, 128
H, O = 1024, 1000

NCH = 2
BCH = B // NCH

NC, NS = 2, 16
NW = NC * NS
ROWS_PER_W = BCH // NW
CB = 16
NCHUNK = ROWS_PER_W // CB
CROWS = CB * L
NLANE = D // 16


def _pool_body(off, xf_hbm, e_hbm, out_hbm,
               idx0, idx1, rows0, rows1, acc_v,
               sem0, sem1, isem0, isem1):
    wid = lax.axis_index("s") * NC + lax.axis_index("c")
    base = wid * ROWS_PER_W
    idx = (idx0, idx1)
    rows = (rows0, rows1)
    sem = (sem0, sem1)
    isem = (isem0, isem1)

    def xsrc(c):
        return xf_hbm.at[pl.ds((off + base + c * CB) * L, CROWS)]

    def stage_idx(c):
        pltpu.async_copy(xsrc(c), idx[c % 2], isem[c % 2])

    def gather(c):
        b = c % 2
        pltpu.make_async_copy(xsrc(c), idx[b], isem[b]).wait()
        pltpu.async_copy(e_hbm.at[idx[b]], rows[b], sem[b])

    def accum(c):
        b = c % 2
        pltpu.make_async_copy(e_hbm.at[idx[b]], rows[b], sem[b]).wait()
        rv = rows[b]

        def row(r, carry):
            accs = [rv[r * L, pl.ds(d * 16, 16)] for d in range(NLANE)]
            for l in range(1, L):
                for d in range(NLANE):
                    accs[d] = accs[d] + rv[r * L + l, pl.ds(d * 16, 16)]
            for d in range(NLANE):
                acc_v[r, pl.ds(d * 16, 16)] = accs[d]
            return carry

        lax.fori_loop(0, CB, row, 0)
        pltpu.sync_copy(acc_v, out_hbm.at[pl.ds(base + c * CB, CB)])

    stage_idx(0)
    gather(0)
    stage_idx(1)
    for c in range(NCHUNK):
        if c + 1 < NCHUNK:
            gather(c + 1)
        accum(c)
        if c + 2 < NCHUNK:
            stage_idx(c + 2)


def _make_pool(ci):
    return functools.partial(
        pl.kernel,
        out_type=jax.ShapeDtypeStruct((BCH, D), jnp.float32),
        mesh=plsc.VectorSubcoreMesh(core_axis_name="c", subcore_axis_name="s"),
        scratch_types=[
            pltpu.VMEM((CROWS,), jnp.int32),
            pltpu.VMEM((CROWS,), jnp.int32),
            pltpu.VMEM((CROWS, D), jnp.float32),
            pltpu.VMEM((CROWS, D), jnp.float32),
            pltpu.VMEM((CB, D), jnp.float32),
            pltpu.SemaphoreType.DMA,
            pltpu.SemaphoreType.DMA,
            pltpu.SemaphoreType.DMA,
            pltpu.SemaphoreType.DMA,
        ],
    )(functools.partial(_pool_body, ci * BCH))


_POOLS = [_make_pool(ci) for ci in range(NCH)]

BM = 512
NBLK = BCH // BM


def _mlp_body(x_ref, hs_ref, w1t_ref, b1_ref, w2t_ref, b2_ref, o_ref):
    cnt = jnp.sum((x_ref[...] != 0).astype(jnp.float32), axis=1, keepdims=True)
    h = hs_ref[...] / jnp.maximum(cnt, 1.0)
    h1t = lax.dot_general(w1t_ref[...], h, (((1,), (1,)), ((), ())),
                          preferred_element_type=jnp.float32)
    h1t = jnp.maximum(h1t + b1_ref[...], 0.0)
    o_ref[...] = jnp.dot(w2t_ref[...], h1t,
                         preferred_element_type=jnp.float32) + b2_ref[...]


def _mlp_chunk(ci, x, hs, W1t, b1c, W2t, b2c, acc):
    base = ci * NBLK
    in_specs = [
        pl.BlockSpec((BM, L), lambda i: (base + i, 0)),
        pl.BlockSpec((BM, D), lambda i: (i, 0)),
        pl.BlockSpec((H, D), lambda i: (0, 0)),
        pl.BlockSpec((H, 1), lambda i: (0, 0)),
        pl.BlockSpec((O, H), lambda i: (0, 0)),
        pl.BlockSpec((O, 1), lambda i: (0, 0)),
    ]
    args = [x, hs, W1t, b1c, W2t, b2c]
    kwargs = {}
    body = _mlp_body
    if acc is not None:
        in_specs.append(pl.BlockSpec(memory_space=pl.ANY))
        args.append(acc)
        kwargs["input_output_aliases"] = {6: 0}
        body = lambda x_r, hs_r, w1_r, b1_r, w2_r, b2_r, a_r, o_r: (
            _mlp_body(x_r, hs_r, w1_r, b1_r, w2_r, b2_r, o_r))
    return pl.pallas_call(
        body,
        grid=(NBLK,),
        in_specs=in_specs,
        out_specs=pl.BlockSpec((O, BM), lambda i: (0, base + i)),
        out_shape=jax.ShapeDtypeStruct((O, B), jnp.float32),
        **kwargs,
    )(*args)


def kernel(x, E, W1, b1, W2, b2):
    x = x.astype(jnp.int32)
    xf = x.reshape(-1)
    W1t = W1.T
    W2t = W2.T
    b1c = b1.reshape(H, 1)
    b2c = b2.reshape(O, 1)
    hs = [_POOLS[ci](xf, E) for ci in range(NCH)]
    outt = None
    for ci in range(NCH):
        outt = _mlp_chunk(ci, x, hs[ci], W1t, b1c, W2t, b2c, outt)
    return outt.T
